# Initial kernel scaffold; baseline (speedup 1.0000x reference)
import jax
import jax.numpy as jnp
from jax.experimental import pallas as pl


def kernel(x, neuron_weights, link_weights_a, link_weights_b, link_mask_a, link_mask_b):
    raise NotImplementedError("write your pallas kernel here")


# speedup vs baseline: 1.0646x; 1.0646x over previous
import functools

import jax
import jax.numpy as jnp
from jax import lax
from jax.experimental import pallas as pl
from jax.experimental.pallas import tpu as pltpu
from jax.experimental.pallas import tpu_sc as plsc

IN_DIM = 2048
OUT_DIM = 2048
BATCH = 8192

_C0 = (0., 0., 0., 0., 0., 0., 0., 0., 1., 1., 1., 1., 1., 1., 1., 1.)
_C1 = (0., 0., 1., 1., 0., 0., 1., 1., -1., -1., 0., 0., -1., -1., 0., 0.)
_C2 = (0., 0., 0., 0., 1., 1., 1., 1., -1., -1., -1., -1., 0., 0., 0., 0.)
_C3 = (0., 1., -1., 0., -1., 0., -2., -1., 1., 2., 0., 1., 0., 1., -1., 0.)


def _first_argmax(w, iota, sentinel):
    mx = jnp.max(w, axis=1, keepdims=True)
    return jnp.min(jnp.where(w == mx, iota, sentinel), axis=1)


def _idx_coeff_body(lwa_ref, lwb_ref, maf_ref, mbf_ref, nw_ref, idx_ref, coef_ref):
    rows = lwa_ref.shape[0]
    iota_in = lax.broadcasted_iota(jnp.int32, (rows, IN_DIM), 1)
    wa = jnp.where(maf_ref[...] != 0.0, lwa_ref[...], -1e30)
    wb = jnp.where(mbf_ref[...] != 0.0, lwb_ref[...], -1e30)
    ia = _first_argmax(wa, iota_in, IN_DIM)
    ib = _first_argmax(wb, iota_in, IN_DIM)
    zeros_i = jnp.zeros((6, rows), jnp.int32)
    idx_ref[...] = jnp.concatenate([ia[None, :], ib[None, :], zeros_i], axis=0)

    iota_g = lax.broadcasted_iota(jnp.int32, (rows, 16), 1)
    g = _first_argmax(nw_ref[...], iota_g, 16)[None, :]
    c0 = jnp.zeros_like(g, dtype=jnp.float32)
    c1, c2, c3 = c0, c0, c0
    for k in range(16):
        sel = (g == k).astype(jnp.float32)
        c0 = c0 + sel * _C0[k]
        c1 = c1 + sel * _C1[k]
        c2 = c2 + sel * _C2[k]
        c3 = c3 + sel * _C3[k]
    zeros_f = jnp.zeros((4, rows), jnp.float32)
    coef_ref[...] = jnp.concatenate([c0, c1, c2, c3, zeros_f], axis=0)


def _transpose_body(x_ref, xt_ref):
    xt_ref[...] = jnp.swapaxes(x_ref[...], 0, 1)


def _gate_body(a_ref, b_ref, coef_ref, out_ref):
    at = jnp.swapaxes(a_ref[...], 0, 1)
    bt = jnp.swapaxes(b_ref[...], 0, 1)
    c0 = coef_ref[0, :][None, :]
    c1 = coef_ref[1, :][None, :]
    c2 = coef_ref[2, :][None, :]
    c3 = coef_ref[3, :][None, :]
    out_ref[...] = c0 + c1 * at + c2 * bt + c3 * (at * bt)


def _sc_gather(xT, idx_flat):
    mesh = plsc.VectorSubcoreMesh(core_axis_name="c", subcore_axis_name="s")

    @functools.partial(
        pl.kernel,
        mesh=mesh,
        out_type=jax.ShapeDtypeStruct((2 * OUT_DIM, BATCH), jnp.float32),
        scratch_types=[
            pltpu.VMEM((128,), jnp.int32),
            pltpu.VMEM((8, BATCH), jnp.float32),
            pltpu.SemaphoreType.DMA,
        ],
    )
    def k(xT_hbm, idx_hbm, out_hbm, idx_v, rows_v, sem):
        wid = lax.axis_index("s") * 2 + lax.axis_index("c")
        base = wid * 128
        pltpu.sync_copy(idx_hbm.at[pl.ds(base, 128)], idx_v)

        @pl.loop(0, 16)
        def _(c):
            pltpu.async_copy(
                xT_hbm.at[idx_v.at[pl.ds(c * 8, 8)]], rows_v, sem
            ).wait()
            pltpu.sync_copy(rows_v, out_hbm.at[pl.ds(base + c * 8, 8)])

    return k(xT, idx_flat)


def _stage1(lwa, lwb, maf, mbf, nw, interpret=False):
    blk = 256
    grid = OUT_DIM // blk
    return pl.pallas_call(
        _idx_coeff_body,
        grid=(grid,),
        in_specs=[
            pl.BlockSpec((blk, IN_DIM), lambda g: (g, 0)),
            pl.BlockSpec((blk, IN_DIM), lambda g: (g, 0)),
            pl.BlockSpec((blk, IN_DIM), lambda g: (g, 0)),
            pl.BlockSpec((blk, IN_DIM), lambda g: (g, 0)),
            pl.BlockSpec((blk, 16), lambda g: (g, 0)),
        ],
        out_specs=[
            pl.BlockSpec((8, blk), lambda g: (0, g)),
            pl.BlockSpec((8, blk), lambda g: (0, g)),
        ],
        out_shape=[
            jax.ShapeDtypeStruct((8, OUT_DIM), jnp.int32),
            jax.ShapeDtypeStruct((8, OUT_DIM), jnp.float32),
        ],
        interpret=interpret,
    )(lwa, lwb, maf, mbf, nw)


def _stage2(x, interpret=False):
    rb, cb = 512, 512
    return pl.pallas_call(
        _transpose_body,
        grid=(BATCH // rb, IN_DIM // cb),
        in_specs=[pl.BlockSpec((rb, cb), lambda i, j: (i, j))],
        out_specs=pl.BlockSpec((cb, rb), lambda i, j: (j, i)),
        out_shape=jax.ShapeDtypeStruct((IN_DIM, BATCH), jnp.float32),
        interpret=interpret,
    )(x)


def _stage4(gT, coeffs, interpret=False):
    nb, bb = 256, 1024
    return pl.pallas_call(
        _gate_body,
        grid=(OUT_DIM // nb, BATCH // bb),
        in_specs=[
            pl.BlockSpec((nb, bb), lambda j, i: (j, i)),
            pl.BlockSpec((nb, bb), lambda j, i: (j + OUT_DIM // nb, i)),
            pl.BlockSpec((8, nb), lambda j, i: (0, j)),
        ],
        out_specs=pl.BlockSpec((bb, nb), lambda j, i: (i, j)),
        out_shape=jax.ShapeDtypeStruct((BATCH, OUT_DIM), jnp.float32),
        interpret=interpret,
    )(gT, gT, coeffs)


def kernel(x, neuron_weights, link_weights_a, link_weights_b, link_mask_a, link_mask_b):
    maf = link_mask_a.astype(jnp.float32)
    mbf = link_mask_b.astype(jnp.float32)
    idxmat, coeffs = _stage1(link_weights_a, link_weights_b, maf, mbf, neuron_weights)
    idx_flat = jnp.concatenate([idxmat[0], idxmat[1]])
    xT = _stage2(x)
    gT = _sc_gather(xT, idx_flat)
    return _stage4(gT, coeffs)
